# Initial kernel scaffold; baseline (speedup 1.0000x reference)
import jax
import jax.numpy as jnp
from jax.experimental import pallas as pl


def kernel(edge_index, weights):
    raise NotImplementedError("write your pallas kernel here")


# speedup vs baseline: 81.8943x; 81.8943x over previous
import functools

import jax
import jax.numpy as jnp
from jax import lax
from jax.experimental import pallas as pl
from jax.experimental.pallas import tpu as pltpu
from jax.experimental.pallas import tpu_sc as plsc

N_EDGES = 3_200_000
N_ROWS = 100_000

NC = 2
NS = 16
LANES = 128
EROWS = 25_600
PAD_EDGES = EROWS * LANES
ROWS_TILE = EROWS // (NC * NS)
ROWS_PAD = 102_400
ACC_TILE = ROWS_PAD // NS
K = 8
CHUNKS = ROWS_TILE // K

_mesh = plsc.VectorSubcoreMesh(
    core_axis_name="c", subcore_axis_name="s", num_cores=NC, num_subcores=NS
)


@functools.partial(
    pl.kernel,
    out_type=jax.ShapeDtypeStruct((NC, ROWS_PAD), jnp.float32),
    mesh=_mesh,
    scratch_types=[
        pltpu.VMEM((K, LANES), jnp.int32),
        pltpu.VMEM((K, LANES), jnp.float32),
        pltpu.VMEM((ACC_TILE,), jnp.float32),
        pltpu.VMEM_SHARED((ROWS_PAD,), jnp.float32),
        pltpu.SemaphoreType.DMA,
    ],
)
def _segment_sum_sc(rows_hbm, w_hbm, partial_hbm, idx_buf, w_buf, zbuf, acc, sem):
    c = lax.axis_index("c")
    s = lax.axis_index("s")

    def _zero(i, _):
        zbuf[pl.ds(i * 16, 16)] = jnp.zeros((16,), jnp.float32)
        return 0

    lax.fori_loop(0, ACC_TILE // 16, _zero, 0)
    pltpu.sync_copy(zbuf, acc.at[pl.ds(s * ACC_TILE, ACC_TILE)])
    plsc.subcore_barrier()

    row_base = (c * NS + s) * ROWS_TILE

    def _chunk(g, _):
        r0 = row_base + g * K
        pltpu.sync_copy(rows_hbm.at[pl.ds(r0, K)], idx_buf)
        pltpu.sync_copy(w_hbm.at[pl.ds(r0, K)], w_buf)
        cps = [
            pltpu.async_copy(w_buf.at[k], acc.at[idx_buf.at[k]], sem, add=True)
            for k in range(K)
        ]
        for cp in cps:
            cp.wait()
        return 0

    lax.fori_loop(0, CHUNKS, _chunk, 0)
    plsc.subcore_barrier()

    pltpu.sync_copy(
        acc.at[pl.ds(s * ACC_TILE, ACC_TILE)],
        partial_hbm.at[c, pl.ds(s * ACC_TILE, ACC_TILE)],
    )


def _finalize_tc(p_ref, inv_ref, rs_ref):
    p = p_ref[0] + p_ref[1]
    inv = 1.0 / (p + 1e-8)
    inv_ref[...] = inv
    rs_ref[...] = p * inv


_finalize = pl.pallas_call(
    _finalize_tc,
    out_shape=[
        jax.ShapeDtypeStruct((ROWS_PAD // 128, 128), jnp.float32),
        jax.ShapeDtypeStruct((ROWS_PAD // 128, 128), jnp.float32),
    ],
)


@functools.partial(
    pl.kernel,
    out_type=jax.ShapeDtypeStruct((EROWS, LANES), jnp.float32),
    mesh=_mesh,
    compiler_params=pltpu.CompilerParams(needs_layout_passes=False),
    scratch_types=[
        pltpu.VMEM((ROWS_PAD,), jnp.float32),
        pltpu.VMEM((K, LANES), jnp.int32),
        pltpu.VMEM((K, LANES), jnp.float32),
        pltpu.VMEM((K, LANES), jnp.float32),
    ],
)
def _gather_mul_sc(rows_hbm, w_hbm, inv_hbm, wn_hbm, inv_vmem, idx_buf, w_buf, out_buf):
    c = lax.axis_index("c")
    s = lax.axis_index("s")

    pltpu.sync_copy(inv_hbm, inv_vmem)

    row_base = (c * NS + s) * ROWS_TILE

    def _chunk(g, _):
        r0 = row_base + g * K
        pltpu.sync_copy(rows_hbm.at[pl.ds(r0, K)], idx_buf)
        pltpu.sync_copy(w_hbm.at[pl.ds(r0, K)], w_buf)
        for k in range(K):
            for j in range(LANES // 16):
                sl = pl.ds(j * 16, 16)
                idx16 = idx_buf[k, sl]
                g16 = plsc.load_gather(inv_vmem, [idx16])
                out_buf[k, sl] = w_buf[k, sl] * g16
        pltpu.sync_copy(out_buf, wn_hbm.at[pl.ds(r0, K)])
        return 0

    lax.fori_loop(0, CHUNKS, _chunk, 0)


def kernel(edge_index, weights):
    rows = edge_index[1]
    pad = PAD_EDGES - N_EDGES
    rows_p = jnp.concatenate([rows, jnp.full((pad,), N_ROWS, jnp.int32)])
    w_p = jnp.concatenate([weights, jnp.zeros((pad,), jnp.float32)])
    rows2d = rows_p.reshape(EROWS, LANES)
    w2d = w_p.reshape(EROWS, LANES)

    partial = _segment_sum_sc(rows2d, w2d)
    inv_norm, row_sums = _finalize(partial.reshape(NC, ROWS_PAD // 128, 128))
    wn = _gather_mul_sc(rows2d, w2d, inv_norm.reshape(ROWS_PAD))
    return wn.reshape(-1)[:N_EDGES], row_sums.reshape(-1)[:N_ROWS]


# speedup vs baseline: 157.4564x; 1.9227x over previous
import functools

import jax
import jax.numpy as jnp
from jax import lax
from jax.experimental import pallas as pl
from jax.experimental.pallas import tpu as pltpu
from jax.experimental.pallas import tpu_sc as plsc

N_EDGES = 3_200_000
N_ROWS = 100_000

NC = 2
NS = 16
LANES = 128
EROWS = 25_600
PAD_EDGES = EROWS * LANES
ROWS_TILE = EROWS // (NC * NS)
ROWS_PAD = 102_400
ACC_TILE = ROWS_PAD // NS
K = 16
CHUNKS = ROWS_TILE // K

_mesh = plsc.VectorSubcoreMesh(
    core_axis_name="c", subcore_axis_name="s", num_cores=NC, num_subcores=NS
)


@functools.partial(
    pl.kernel,
    out_type=jax.ShapeDtypeStruct((NC, ROWS_PAD), jnp.float32),
    mesh=_mesh,
    scratch_types=[
        pltpu.VMEM((2, K, LANES), jnp.int32),
        pltpu.VMEM((2, K, LANES), jnp.float32),
        pltpu.VMEM((ACC_TILE,), jnp.float32),
        pltpu.VMEM_SHARED((ROWS_PAD,), jnp.float32),
        pltpu.SemaphoreType.DMA,
        pltpu.SemaphoreType.DMA,
        pltpu.SemaphoreType.DMA,
    ],
)
def _segment_sum_sc(
    rows_hbm, w_hbm, partial_hbm, idx_buf, w_buf, zbuf, acc, ld0, ld1, sc_sem
):
    c = lax.axis_index("c")
    s = lax.axis_index("s")
    ld = (ld0, ld1)

    def _zero(i, _):
        zbuf[pl.ds(i * 16, 16)] = jnp.zeros((16,), jnp.float32)
        return 0

    lax.fori_loop(0, ACC_TILE // 16, _zero, 0)
    pltpu.sync_copy(zbuf, acc.at[pl.ds(s * ACC_TILE, ACC_TILE)])
    plsc.subcore_barrier()

    row_base = (c * NS + s) * ROWS_TILE

    def _load(g, b):
        r0 = row_base + g * K
        pltpu.async_copy(rows_hbm.at[pl.ds(r0, K)], idx_buf.at[b], ld[b])
        pltpu.async_copy(w_hbm.at[pl.ds(r0, K)], w_buf.at[b], ld[b])

    def _drain_load(b):
        pltpu.make_async_copy(rows_hbm.at[pl.ds(0, K)], idx_buf.at[b], ld[b]).wait()
        pltpu.make_async_copy(w_hbm.at[pl.ds(0, K)], w_buf.at[b], ld[b]).wait()

    _load(0, 0)

    def _pair(gg, _):
        for b in range(2):
            g = gg * 2 + b

            @pl.when(g + 1 < CHUNKS)
            def _():
                _load(g + 1, 1 - b)

            _drain_load(b)
            cps = [
                pltpu.async_copy(
                    w_buf.at[b, k], acc.at[idx_buf.at[b, k]], sc_sem, add=True
                )
                for k in range(K)
            ]
            for cp in cps:
                cp.wait()
        return 0

    lax.fori_loop(0, CHUNKS // 2, _pair, 0)
    plsc.subcore_barrier()

    pltpu.sync_copy(
        acc.at[pl.ds(s * ACC_TILE, ACC_TILE)],
        partial_hbm.at[c, pl.ds(s * ACC_TILE, ACC_TILE)],
    )


def _finalize_tc(p_ref, inv_ref, rs_ref):
    p = p_ref[0] + p_ref[1]
    inv = 1.0 / (p + 1e-8)
    inv_ref[...] = inv
    rs_ref[...] = p * inv


_finalize = pl.pallas_call(
    _finalize_tc,
    out_shape=[
        jax.ShapeDtypeStruct((ROWS_PAD // 128, 128), jnp.float32),
        jax.ShapeDtypeStruct((ROWS_PAD // 128, 128), jnp.float32),
    ],
)


@functools.partial(
    pl.kernel,
    out_type=jax.ShapeDtypeStruct((EROWS, LANES), jnp.float32),
    mesh=_mesh,
    compiler_params=pltpu.CompilerParams(needs_layout_passes=False),
    scratch_types=[
        pltpu.VMEM((ROWS_PAD,), jnp.float32),
        pltpu.VMEM((2, K, LANES), jnp.int32),
        pltpu.VMEM((2, K, LANES), jnp.float32),
        pltpu.VMEM((2, K, LANES), jnp.float32),
        pltpu.SemaphoreType.DMA,
        pltpu.SemaphoreType.DMA,
        pltpu.SemaphoreType.DMA,
        pltpu.SemaphoreType.DMA,
    ],
)
def _gather_mul_sc(
    rows_hbm, w_hbm, inv_hbm, wn_hbm,
    inv_vmem, idx_buf, w_buf, out_buf, ld0, ld1, st0, st1,
):
    c = lax.axis_index("c")
    s = lax.axis_index("s")
    ld = (ld0, ld1)
    st = (st0, st1)

    pltpu.sync_copy(inv_hbm, inv_vmem)

    row_base = (c * NS + s) * ROWS_TILE

    def _load(g, b):
        r0 = row_base + g * K
        pltpu.async_copy(rows_hbm.at[pl.ds(r0, K)], idx_buf.at[b], ld[b])
        pltpu.async_copy(w_hbm.at[pl.ds(r0, K)], w_buf.at[b], ld[b])

    def _drain_load(b):
        pltpu.make_async_copy(rows_hbm.at[pl.ds(0, K)], idx_buf.at[b], ld[b]).wait()
        pltpu.make_async_copy(w_hbm.at[pl.ds(0, K)], w_buf.at[b], ld[b]).wait()

    def _drain_store(b):
        pltpu.make_async_copy(out_buf.at[b], wn_hbm.at[pl.ds(0, K)], st[b]).wait()

    _load(0, 0)

    def _pair(gg, _):
        for b in range(2):
            g = gg * 2 + b

            @pl.when(g + 1 < CHUNKS)
            def _():
                _load(g + 1, 1 - b)

            _drain_load(b)

            @pl.when(gg >= 1)
            def _():
                _drain_store(b)

            for k in range(K):
                for j in range(LANES // 16):
                    sl = pl.ds(j * 16, 16)
                    idx16 = idx_buf[b, k, sl]
                    g16 = plsc.load_gather(inv_vmem, [idx16])
                    out_buf[b, k, sl] = w_buf[b, k, sl] * g16
            r0 = row_base + g * K
            pltpu.async_copy(out_buf.at[b], wn_hbm.at[pl.ds(r0, K)], st[b])
        return 0

    lax.fori_loop(0, CHUNKS // 2, _pair, 0)
    _drain_store(0)
    _drain_store(1)


def kernel(edge_index, weights):
    rows = edge_index[1]
    pad = PAD_EDGES - N_EDGES
    rows_p = jnp.concatenate([rows, jnp.full((pad,), N_ROWS, jnp.int32)])
    w_p = jnp.concatenate([weights, jnp.zeros((pad,), jnp.float32)])
    rows2d = rows_p.reshape(EROWS, LANES)
    w2d = w_p.reshape(EROWS, LANES)

    partial = _segment_sum_sc(rows2d, w2d)
    inv_norm, row_sums = _finalize(partial.reshape(NC, ROWS_PAD // 128, 128))
    wn = _gather_mul_sc(rows2d, w2d, inv_norm.reshape(ROWS_PAD))
    return wn.reshape(-1)[:N_EDGES], row_sums.reshape(-1)[:N_ROWS]
